# baseline (speedup 1.0000x reference)
import functools

import jax
import jax.numpy as jnp
from jax import lax
from jax.experimental import pallas as pl
from jax.experimental.pallas import tpu as pltpu
from jax.experimental.pallas import tpu_sc as plsc

NUM_CLASSES = 80
D = 256
NC, NS = 2, 16
NW = NC * NS
BSZ, N = 1024, 300
NPAD = 304
KSPLIT = 4
BK = BSZ // KSPLIT
B_PER_W = BK // NW
OFFS = (0, 56, 104, 152, 200, 248)
SIZES = (56, 48, 48, 48, 48, 56)
NSLOT = 6
NREP = 64
TB = 8


def _worker(table_hbm, idx_hbm, out_hbm, idx_v, buf, gsems, wsems):
    wid = lax.axis_index("s") * NC + lax.axis_index("c")
    b0 = wid * B_PER_W
    base = pl.multiple_of(b0 * NPAD, 8)
    pltpu.sync_copy(idx_hbm.at[pl.ds(base, B_PER_W * NPAD)], idx_v)

    def start(g, j):
        off = pl.multiple_of(g * NPAD + OFFS[j], 8)
        pltpu.async_copy(
            table_hbm.at[idx_v.at[pl.ds(off, SIZES[j])]],
            buf.at[j, pl.ds(0, SIZES[j])], gsems[j])

    def wait_gather(j):
        pltpu.make_async_copy(
            table_hbm.at[idx_v.at[pl.ds(0, SIZES[j])]],
            buf.at[j, pl.ds(0, SIZES[j])], gsems[j]).wait()

    def write(g, j):
        pltpu.async_copy(
            buf.at[j, pl.ds(0, SIZES[j])],
            out_hbm.at[b0 + g, pl.ds(OFFS[j], SIZES[j])], wsems[j])

    def wait_write(g, j):
        pltpu.make_async_copy(
            buf.at[j, pl.ds(0, SIZES[j])],
            out_hbm.at[b0 + g, pl.ds(OFFS[j], SIZES[j])], wsems[j]).wait()

    for j in range(3):
        start(0, j)

    for j in range(NSLOT):
        wait_gather(j)
        write(0, j)
        if j < 3:
            start(0, j + 3)
        else:
            wait_write(0, j - 3)
            start(1, j - 3)

    def outer(g, carry):
        for j in range(NSLOT):
            wait_gather(j)
            write(g, j)
            if j < 3:
                wait_write(g - 1, j + 3)
                start(g, j + 3)
            else:
                wait_write(g, j - 3)
                start(g + 1, j - 3)
        return carry

    lax.fori_loop(1, B_PER_W - 1, outer, 0)

    g_last = B_PER_W - 1
    for j in range(NSLOT):
        wait_gather(j)
        write(g_last, j)
        if j < 3:
            wait_write(g_last - 1, j + 3)
            start(g_last, j + 3)
    for j in range(NSLOT):
        wait_write(g_last, j)


_sc_gather = functools.partial(
    pl.kernel,
    out_type=jax.ShapeDtypeStruct((BK, NPAD, D), jnp.float32),
    mesh=plsc.VectorSubcoreMesh(core_axis_name="c", subcore_axis_name="s"),
    scratch_types=[
        pltpu.VMEM((B_PER_W * NPAD,), jnp.int32),
        pltpu.VMEM((NSLOT, max(SIZES), D), jnp.float32),
        [pltpu.SemaphoreType.DMA] * NSLOT,
        [pltpu.SemaphoreType.DMA] * NSLOT,
    ],
)(_worker)


def _tc_first(src, out):
    out[...] = src[:, :N, :]


def _tc_chained(src, prev, out):
    del prev
    out[...] = src[:, :N, :]


def _tc_fixup(i, sc_part, prev):
    grid = (BK // TB,)
    base = i * (BK // TB)
    in_spec = pl.BlockSpec((TB, NPAD, D), lambda g: (g, 0, 0))
    out_spec = pl.BlockSpec((TB, N, D), lambda g: (base + g, 0, 0))
    out_shape = jax.ShapeDtypeStruct((BSZ, N, D), jnp.float32)
    if prev is None:
        return pl.pallas_call(
            _tc_first, grid=grid, in_specs=[in_spec], out_specs=out_spec,
            out_shape=out_shape)(sc_part)
    return pl.pallas_call(
        _tc_chained, grid=grid,
        in_specs=[in_spec, pl.BlockSpec(memory_space=pl.ANY)],
        out_specs=out_spec, out_shape=out_shape,
        input_output_aliases={1: 0})(sc_part, prev)


def kernel(labels, label_embed_table):
    nrows = label_embed_table.shape[0]
    ones = jnp.ones((nrows, 1), label_embed_table.dtype)
    table = jnp.concatenate([label_embed_table, ones], axis=-1)
    table_rep = jnp.tile(table, (NREP, 1))
    labels_p = jnp.pad(labels, ((0, 0), (0, NPAD - N))).reshape(
        KSPLIT, BK * NPAD)
    rot = (jnp.arange(BK * NPAD, dtype=jnp.int32) % NREP) * nrows
    out = None
    for i in range(KSPLIT):
        sc_i = _sc_gather(table_rep, labels_p[i] + rot)
        out = _tc_fixup(i, sc_i, out)
    return out


# speedup vs baseline: 1.4445x; 1.4445x over previous
import functools

import jax
import jax.numpy as jnp
from jax import lax
from jax.experimental import pallas as pl
from jax.experimental.pallas import tpu as pltpu
from jax.experimental.pallas import tpu_sc as plsc

NUM_CLASSES = 80
D = 256
NC, NS = 2, 16
NW = NC * NS
BSZ, N = 1024, 300
NSC = 296
NPAD = 304
B_PER_W = BSZ // NW
OFFS = (0, 56, 104, 152, 200, 248)
SIZES = (56, 48, 48, 48, 48, 48)
NSLOT = 6
NREP = 64
TBB = 64


def _worker(table_hbm, idx_hbm, out_hbm, idx_v, buf, gsems, wsems):
    wid = lax.axis_index("s") * NC + lax.axis_index("c")
    b0 = wid * B_PER_W
    base = pl.multiple_of(b0 * NPAD, 8)
    pltpu.sync_copy(idx_hbm.at[pl.ds(base, B_PER_W * NPAD)], idx_v)

    def start(g, j):
        off = pl.multiple_of(g * NPAD + OFFS[j], 8)
        pltpu.async_copy(
            table_hbm.at[idx_v.at[pl.ds(off, SIZES[j])]],
            buf.at[j, pl.ds(0, SIZES[j])], gsems[j])

    def wait_gather(j):
        pltpu.make_async_copy(
            table_hbm.at[idx_v.at[pl.ds(0, SIZES[j])]],
            buf.at[j, pl.ds(0, SIZES[j])], gsems[j]).wait()

    def write(g, j):
        pltpu.async_copy(
            buf.at[j, pl.ds(0, SIZES[j])],
            out_hbm.at[b0 + g, pl.ds(OFFS[j], SIZES[j])], wsems[j])

    def wait_write(g, j):
        pltpu.make_async_copy(
            buf.at[j, pl.ds(0, SIZES[j])],
            out_hbm.at[b0 + g, pl.ds(OFFS[j], SIZES[j])], wsems[j]).wait()

    for j in range(3):
        start(0, j)

    for j in range(NSLOT):
        wait_gather(j)
        write(0, j)
        if j < 3:
            start(0, j + 3)
        else:
            wait_write(0, j - 3)
            start(1, j - 3)

    def outer(g, carry):
        for j in range(NSLOT):
            wait_gather(j)
            write(g, j)
            if j < 3:
                wait_write(g - 1, j + 3)
                start(g, j + 3)
            else:
                wait_write(g, j - 3)
                start(g + 1, j - 3)
        return carry

    lax.fori_loop(1, B_PER_W - 1, outer, 0)

    g_last = B_PER_W - 1
    for j in range(NSLOT):
        wait_gather(j)
        write(g_last, j)
        if j < 3:
            wait_write(g_last - 1, j + 3)
            start(g_last, j + 3)
    for j in range(NSLOT):
        wait_write(g_last, j)


_sc_gather = functools.partial(
    pl.kernel,
    out_type=jax.ShapeDtypeStruct((BSZ, N, D), jnp.float32),
    mesh=plsc.VectorSubcoreMesh(core_axis_name="c", subcore_axis_name="s"),
    scratch_types=[
        pltpu.VMEM((B_PER_W * NPAD,), jnp.int32),
        pltpu.VMEM((NSLOT, max(SIZES), D), jnp.float32),
        [pltpu.SemaphoreType.DMA] * NSLOT,
        [pltpu.SemaphoreType.DMA] * NSLOT,
    ],
)(_worker)


NTAIL = 8


def _tc_tail_body(lbl_ref, table_ref, sc_ref, out_ref):
    del sc_ref
    nrows = table_ref.shape[0]
    tab = table_ref[...]
    iota = lax.broadcasted_iota(jnp.int32, (TBB, nrows), 1)
    for t in range(NTAIL):
        oh = (iota == lbl_ref[:, t][:, None]).astype(jnp.float32)
        out_ref[:, t, :] = lax.dot_general(
            oh, tab, (((1,), (0,)), ((), ())),
            preferred_element_type=jnp.float32)


def _tc_tail(labels_tail, table, sc_out):
    return pl.pallas_call(
        _tc_tail_body,
        grid=(BSZ // TBB,),
        in_specs=[
            pl.BlockSpec((TBB, NTAIL), lambda g: (g, 0)),
            pl.BlockSpec(table.shape, lambda g: (0, 0)),
            pl.BlockSpec(memory_space=pl.ANY),
        ],
        out_specs=pl.BlockSpec((TBB, NTAIL, D), lambda g: (g, NSC // NTAIL, 0)),
        out_shape=jax.ShapeDtypeStruct((BSZ, N, D), jnp.float32),
        input_output_aliases={2: 0},
    )(labels_tail, table, sc_out)


def kernel(labels, label_embed_table):
    nrows = label_embed_table.shape[0]
    ones = jnp.ones((nrows, 1), label_embed_table.dtype)
    table = jnp.concatenate([label_embed_table, ones], axis=-1)
    table_rep = jnp.tile(table, (NREP, 1))
    labels_p = jnp.pad(labels, ((0, 0), (0, NPAD - N))).reshape(-1)
    rot = (jnp.arange(labels_p.shape[0], dtype=jnp.int32) % NREP) * nrows
    sc_out = _sc_gather(table_rep, labels_p + rot)
    labels_tail = jnp.pad(labels[:, NSC:], ((0, 0), (0, NTAIL - (N - NSC))))
    return _tc_tail(labels_tail, table, sc_out)
